# baseline (speedup 1.0000x reference)
import functools

import jax
import jax.numpy as jnp
from jax import lax
from jax.experimental import pallas as pl
from jax.experimental.pallas import tpu as pltpu
from jax.experimental.pallas import tpu_sc as plsc

V = 1000000
D = 16
F = 26
B = 16384

NC = 2
NS = 16
NW = NC * NS

ROWS = B * F
ROWS_PER_W = ROWS // NW
PIECE = 1664
NPIECE = ROWS_PER_W // PIECE
NCHUNK = PIECE // 16

BLK = 512
NSPLIT = 2


CHV = 1024
NFULL = V // CHV
VMID = NFULL * CHV
VTAIL = V - V % 128


def _sc_transpose(emb_t, tail_flat):
    mesh = plsc.VectorSubcoreMesh(core_axis_name="c", subcore_axis_name="s")

    @functools.partial(
        pl.kernel,
        mesh=mesh,
        compiler_params=pltpu.CompilerParams(use_tc_tiling_on_sc=True,
                                             needs_layout_passes=False),
        out_type=[jax.ShapeDtypeStruct((V * D,), jnp.float32)],
        scratch_types=[
            pltpu.VMEM((D, CHV), jnp.float32),
            pltpu.VMEM((D, CHV), jnp.float32),
            pltpu.VMEM((CHV * D,), jnp.float32),
            pltpu.VMEM((CHV * D,), jnp.float32),
            pltpu.SemaphoreType.DMA,
            pltpu.SemaphoreType.DMA,
            pltpu.SemaphoreType.DMA,
            pltpu.SemaphoreType.DMA,
        ],
    )
    def tr_kernel(src, tail, out, in0, in1, ob0, ob1, si0, si1, so0, so1):
        wid = lax.axis_index("s") * NC + lax.axis_index("c")
        lane = jnp.arange(16, dtype=jnp.int32)
        nfc = (NFULL - wid + NW - 1) // NW
        sins = [si0, si1]
        souts = [so0, so1]
        inbufs = [in0, in1]
        obufs = [ob0, ob1]

        def start_in(k_, b):
            pltpu.async_copy(
                src.at[:, pl.ds((wid + NW * k_) * CHV, CHV)], inbufs[b], sins[b])

        def wait_in(b):
            pltpu.make_async_copy(src.at[:, pl.ds(0, CHV)], inbufs[b],
                                  sins[b]).wait()

        def wait_out(b):
            pltpu.make_async_copy(obufs[b], out.at[pl.ds(0, CHV * D)],
                                  souts[b]).wait()

        base = lane * D

        def transpose_buf(b, n):
            @plsc.parallel_loop(0, n // 16, unroll=4)
            def _jloop(j):
                row = base + j * 256
                for d in range(D):
                    vals = inbufs[b][d, pl.ds(j * 16, 16)]
                    plsc.store_scatter(obufs[b], [row + d], vals)

        def flush(k_, b):
            pltpu.async_copy(
                obufs[b],
                out.at[pl.ds((wid + NW * k_) * CHV * D, CHV * D)], souts[b])

        start_in(0, 0)

        def pbody(p, carry):
            k0 = 2 * p

            @pl.when(k0 + 1 < nfc)
            def _pre1():
                start_in(k0 + 1, 1)

            wait_in(0)

            @pl.when(p >= 1)
            def _dr0():
                wait_out(0)

            transpose_buf(0, CHV)
            flush(k0, 0)

            @pl.when(k0 + 1 < nfc)
            def _second():
                @pl.when(k0 + 2 < nfc)
                def _pre0():
                    start_in(k0 + 2, 0)

                wait_in(1)

                @pl.when(p >= 1)
                def _dr1():
                    wait_out(1)

                transpose_buf(1, CHV)
                flush(k0 + 1, 1)

            return carry

        lax.fori_loop(0, (nfc + 1) // 2, pbody, 0, unroll=False)
        wait_out(0)
        wait_out(1)

        @pl.when(wid == 0)
        def _mid():
            n = VTAIL - VMID
            pltpu.sync_copy(src.at[:, pl.ds(VMID, n)], in0.at[:, pl.ds(0, n)])
            @plsc.parallel_loop(0, n // 16, unroll=2)
            def _jloop(j):
                row = base + j * 256
                for d in range(D):
                    vals = in0[d, pl.ds(j * 16, 16)]
                    plsc.store_scatter(ob0, [row + d], vals)
            pltpu.sync_copy(ob0.at[pl.ds(0, n * D)],
                            out.at[pl.ds(VMID * D, n * D)])

        @pl.when(wid == 1)
        def _tail():
            pltpu.sync_copy(tail, out.at[pl.ds(VTAIL * D, (V - VTAIL) * D)])

    return tr_kernel(emb_t, tail_flat)


def _sc_gather(emb_w, bias16, idx_flat, n_rows):
    mesh = plsc.VectorSubcoreMesh(core_axis_name="c", subcore_axis_name="s")
    rows_per_w = n_rows // NW
    npiece = rows_per_w // PIECE

    @functools.partial(
        pl.kernel,
        mesh=mesh,
        compiler_params=pltpu.CompilerParams(use_tc_tiling_on_sc=False,
                                             needs_layout_passes=False),
        out_type=[
            jax.ShapeDtypeStruct((n_rows, D), jnp.float32),
            jax.ShapeDtypeStruct((n_rows,), jnp.float32),
        ],
        scratch_types=[
            pltpu.VMEM((rows_per_w,), jnp.int32),
            pltpu.VMEM((PIECE, D), jnp.float32),
            pltpu.VMEM((PIECE, 16), jnp.float32),
            pltpu.VMEM((PIECE,), jnp.int32),
            pltpu.VMEM((PIECE,), jnp.float32),
            pltpu.SemaphoreType.DMA,
            pltpu.SemaphoreType.DMA,
        ],
    )
    def gather_kernel(emb_hbm, bias_hbm, idx_hbm, emb_out, bias_out,
                      idx_v, ebuf, brows, idx_d, bvals, sem_e, sem_b):
        wid = lax.axis_index("s") * NC + lax.axis_index("c")
        base = wid * rows_per_w
        pltpu.sync_copy(idx_hbm.at[pl.ds(base, rows_per_w)], idx_v)
        lane = jnp.arange(16, dtype=jnp.int32)
        for p in range(npiece):
            piece_idx = idx_v.at[pl.ds(p * PIECE, PIECE)]
            ce = pltpu.async_copy(emb_hbm.at[piece_idx], ebuf, sem_e)

            @plsc.parallel_loop(0, NCHUNK, unroll=4)
            def _div(k):
                v = idx_v[pl.ds(p * PIECE + k * 16, 16)]
                idx_d[pl.ds(k * 16, 16)] = lax.shift_right_logical(v, 4)
            cb = pltpu.async_copy(bias_hbm.at[idx_d], brows, sem_b)
            cb.wait()

            @plsc.parallel_loop(0, NCHUNK, unroll=4)
            def _sel(k):
                v = idx_v[pl.ds(p * PIECE + k * 16, 16)]
                rows = k * 16 + lane
                vals = plsc.load_gather(brows, [rows, jnp.bitwise_and(v, 15)])
                bvals[pl.ds(k * 16, 16)] = vals
            ce.wait()
            off = base + p * PIECE
            pltpu.sync_copy(ebuf, emb_out.at[pl.ds(off, PIECE)])
            pltpu.sync_copy(bvals, bias_out.at[pl.ds(off, PIECE)])

    return gather_kernel(emb_w, bias16, idx_flat)


def _tc_fm(xg, bg, m, w0, nb):
    def body(w0_ref, x_ref, b_ref, m_ref, o_ref):
        xb = x_ref[...]
        y = jnp.dot(xb, m_ref[...], preferred_element_type=jnp.float32)
        inter = jnp.sum(y * xb, axis=1)
        lin = jnp.sum(b_ref[...], axis=1) + w0_ref[0, 0]
        o_ref[...] = inter + lin

    return pl.pallas_call(
        body,
        grid=(nb // BLK,),
        in_specs=[
            pl.BlockSpec(memory_space=pltpu.SMEM),
            pl.BlockSpec((BLK, F * D), lambda i: (i, 0)),
            pl.BlockSpec((BLK, F), lambda i: (i, 0)),
            pl.BlockSpec((F * D, F * D), lambda i: (0, 0)),
        ],
        out_specs=pl.BlockSpec((BLK,), lambda i: (i,)),
        out_shape=jax.ShapeDtypeStruct((nb,), jnp.float32),
    )(w0, xg, bg, m)


def kernel(x, w0, bias_w, emb_w, fiw):
    idx_flat = x.astype(jnp.int32).reshape(ROWS)
    tail_flat = emb_w[VTAIL:].reshape((V - VTAIL) * D)
    emb_lin, = _sc_transpose(emb_w.T, tail_flat)
    emb2d = emb_lin.reshape(V, D)
    bias16 = bias_w.reshape(V // 16, 16)
    fu = jnp.triu(fiw, 1)
    fsym = fu + fu.T
    eye = jnp.eye(D, dtype=jnp.float32)
    m = ((0.5 * fsym)[:, None, :, None] * eye[None, :, None, :]).reshape(F * D, F * D)
    w01 = w0.reshape(1, 1)
    nh = B // NSPLIT
    rows_h = nh * F
    outs = []
    for h in range(NSPLIT):
        idx_h = lax.slice(idx_flat, (h * rows_h,), ((h + 1) * rows_h,))
        emb_rows, bias_vals = _sc_gather(emb2d, bias16, idx_h, rows_h)
        xg = emb_rows.reshape(nh, F * D)
        bg = bias_vals.reshape(nh, F)
        outs.append(_tc_fm(xg, bg, m, w01, nh))
    return jnp.concatenate(outs)


# speedup vs baseline: 1.0189x; 1.0189x over previous
import functools

import jax
import jax.numpy as jnp
from jax import lax
from jax.experimental import pallas as pl
from jax.experimental.pallas import tpu as pltpu
from jax.experimental.pallas import tpu_sc as plsc

V = 1000000
D = 16
F = 26
B = 16384

NC = 2
NS = 16
NW = NC * NS

ROWS = B * F
ROWS_PER_W = ROWS // NW
PIECE = 1664
NPIECE = ROWS_PER_W // PIECE
NCHUNK = PIECE // 16

BLK = 512
NSPLIT = 2


CHV = 1024
NFULL = V // CHV
VMID = NFULL * CHV
VTAIL = V - V % 128


def _sc_transpose(emb_t, tail_flat):
    mesh = plsc.VectorSubcoreMesh(core_axis_name="c", subcore_axis_name="s")

    @functools.partial(
        pl.kernel,
        mesh=mesh,
        compiler_params=pltpu.CompilerParams(use_tc_tiling_on_sc=True,
                                             needs_layout_passes=False),
        out_type=[jax.ShapeDtypeStruct((V * D,), jnp.float32)],
        scratch_types=[
            pltpu.VMEM((D, CHV), jnp.float32),
            pltpu.VMEM((D, CHV), jnp.float32),
            pltpu.VMEM((CHV * D,), jnp.float32),
            pltpu.VMEM((CHV * D,), jnp.float32),
            pltpu.SemaphoreType.DMA,
            pltpu.SemaphoreType.DMA,
            pltpu.SemaphoreType.DMA,
            pltpu.SemaphoreType.DMA,
        ],
    )
    def tr_kernel(src, tail, out, in0, in1, ob0, ob1, si0, si1, so0, so1):
        wid = lax.axis_index("s") * NC + lax.axis_index("c")
        lane = jnp.arange(16, dtype=jnp.int32)
        nfc = (NFULL - wid + NW - 1) // NW
        sins = [si0, si1]
        souts = [so0, so1]
        inbufs = [in0, in1]
        obufs = [ob0, ob1]

        def start_in(k_, b):
            pltpu.async_copy(
                src.at[:, pl.ds((wid + NW * k_) * CHV, CHV)], inbufs[b], sins[b])

        def wait_in(b):
            pltpu.make_async_copy(src.at[:, pl.ds(0, CHV)], inbufs[b],
                                  sins[b]).wait()

        def wait_out(b):
            pltpu.make_async_copy(obufs[b], out.at[pl.ds(0, CHV * D)],
                                  souts[b]).wait()

        base = lane * D

        def transpose_buf(b, n):
            @plsc.parallel_loop(0, n // 16, unroll=2)
            def _jloop(j):
                row = base + j * 256
                for d in range(D):
                    vals = inbufs[b][d, pl.ds(j * 16, 16)]
                    plsc.store_scatter(obufs[b], [row + d], vals)

        def flush(k_, b):
            pltpu.async_copy(
                obufs[b],
                out.at[pl.ds((wid + NW * k_) * CHV * D, CHV * D)], souts[b])

        start_in(0, 0)

        def pbody(p, carry):
            k0 = 2 * p

            @pl.when(k0 + 1 < nfc)
            def _pre1():
                start_in(k0 + 1, 1)

            wait_in(0)

            @pl.when(p >= 1)
            def _dr0():
                wait_out(0)

            transpose_buf(0, CHV)
            flush(k0, 0)

            @pl.when(k0 + 1 < nfc)
            def _second():
                @pl.when(k0 + 2 < nfc)
                def _pre0():
                    start_in(k0 + 2, 0)

                wait_in(1)

                @pl.when(p >= 1)
                def _dr1():
                    wait_out(1)

                transpose_buf(1, CHV)
                flush(k0 + 1, 1)

            return carry

        lax.fori_loop(0, (nfc + 1) // 2, pbody, 0, unroll=False)
        wait_out(0)
        wait_out(1)

        @pl.when(wid == 0)
        def _mid():
            n = VTAIL - VMID
            pltpu.sync_copy(src.at[:, pl.ds(VMID, n)], in0.at[:, pl.ds(0, n)])
            @plsc.parallel_loop(0, n // 16, unroll=2)
            def _jloop(j):
                row = base + j * 256
                for d in range(D):
                    vals = in0[d, pl.ds(j * 16, 16)]
                    plsc.store_scatter(ob0, [row + d], vals)
            pltpu.sync_copy(ob0.at[pl.ds(0, n * D)],
                            out.at[pl.ds(VMID * D, n * D)])

        @pl.when(wid == 1)
        def _tail():
            pltpu.sync_copy(tail, out.at[pl.ds(VTAIL * D, (V - VTAIL) * D)])

    return tr_kernel(emb_t, tail_flat)


def _sc_gather(emb_w, bias16, idx_flat, n_rows):
    mesh = plsc.VectorSubcoreMesh(core_axis_name="c", subcore_axis_name="s")
    rows_per_w = n_rows // NW
    npiece = rows_per_w // PIECE

    @functools.partial(
        pl.kernel,
        mesh=mesh,
        compiler_params=pltpu.CompilerParams(use_tc_tiling_on_sc=False,
                                             needs_layout_passes=False),
        out_type=[
            jax.ShapeDtypeStruct((n_rows, D), jnp.float32),
            jax.ShapeDtypeStruct((n_rows,), jnp.float32),
        ],
        scratch_types=[
            pltpu.VMEM((rows_per_w,), jnp.int32),
            pltpu.VMEM((PIECE, D), jnp.float32),
            pltpu.VMEM((PIECE, 16), jnp.float32),
            pltpu.VMEM((PIECE,), jnp.int32),
            pltpu.VMEM((PIECE,), jnp.float32),
            pltpu.SemaphoreType.DMA,
            pltpu.SemaphoreType.DMA,
        ],
    )
    def gather_kernel(emb_hbm, bias_hbm, idx_hbm, emb_out, bias_out,
                      idx_v, ebuf, brows, idx_d, bvals, sem_e, sem_b):
        wid = lax.axis_index("s") * NC + lax.axis_index("c")
        base = wid * rows_per_w
        pltpu.sync_copy(idx_hbm.at[pl.ds(base, rows_per_w)], idx_v)
        lane = jnp.arange(16, dtype=jnp.int32)
        for p in range(npiece):
            piece_idx = idx_v.at[pl.ds(p * PIECE, PIECE)]
            ce = pltpu.async_copy(emb_hbm.at[piece_idx], ebuf, sem_e)

            @plsc.parallel_loop(0, NCHUNK, unroll=4)
            def _div(k):
                v = idx_v[pl.ds(p * PIECE + k * 16, 16)]
                idx_d[pl.ds(k * 16, 16)] = lax.shift_right_logical(v, 4)
            cb = pltpu.async_copy(bias_hbm.at[idx_d], brows, sem_b)
            cb.wait()

            @plsc.parallel_loop(0, NCHUNK, unroll=4)
            def _sel(k):
                v = idx_v[pl.ds(p * PIECE + k * 16, 16)]
                rows = k * 16 + lane
                vals = plsc.load_gather(brows, [rows, jnp.bitwise_and(v, 15)])
                bvals[pl.ds(k * 16, 16)] = vals
            ce.wait()
            off = base + p * PIECE
            pltpu.sync_copy(ebuf, emb_out.at[pl.ds(off, PIECE)])
            pltpu.sync_copy(bvals, bias_out.at[pl.ds(off, PIECE)])

    return gather_kernel(emb_w, bias16, idx_flat)


def _tc_fm(xg, bg, m, w0, nb):
    def body(w0_ref, x_ref, b_ref, m_ref, o_ref):
        xb = x_ref[...]
        y = jnp.dot(xb, m_ref[...], preferred_element_type=jnp.float32)
        inter = jnp.sum(y * xb, axis=1)
        lin = jnp.sum(b_ref[...], axis=1) + w0_ref[0, 0]
        o_ref[...] = inter + lin

    return pl.pallas_call(
        body,
        grid=(nb // BLK,),
        in_specs=[
            pl.BlockSpec(memory_space=pltpu.SMEM),
            pl.BlockSpec((BLK, F * D), lambda i: (i, 0)),
            pl.BlockSpec((BLK, F), lambda i: (i, 0)),
            pl.BlockSpec((F * D, F * D), lambda i: (0, 0)),
        ],
        out_specs=pl.BlockSpec((BLK,), lambda i: (i,)),
        out_shape=jax.ShapeDtypeStruct((nb,), jnp.float32),
    )(w0, xg, bg, m)


def kernel(x, w0, bias_w, emb_w, fiw):
    idx_flat = x.astype(jnp.int32).reshape(ROWS)
    tail_flat = emb_w[VTAIL:].reshape((V - VTAIL) * D)
    emb_lin, = _sc_transpose(emb_w.T, tail_flat)
    emb2d = emb_lin.reshape(V, D)
    bias16 = bias_w.reshape(V // 16, 16)
    fu = jnp.triu(fiw, 1)
    fsym = fu + fu.T
    eye = jnp.eye(D, dtype=jnp.float32)
    m = ((0.5 * fsym)[:, None, :, None] * eye[None, :, None, :]).reshape(F * D, F * D)
    w01 = w0.reshape(1, 1)
    nh = B // NSPLIT
    rows_h = nh * F
    outs = []
    for h in range(NSPLIT):
        idx_h = lax.slice(idx_flat, (h * rows_h,), ((h + 1) * rows_h,))
        emb_rows, bias_vals = _sc_gather(emb2d, bias16, idx_h, rows_h)
        xg = emb_rows.reshape(nh, F * D)
        bg = bias_vals.reshape(nh, F)
        outs.append(_tc_fm(xg, bg, m, w01, nh))
    return jnp.concatenate(outs)
